# Initial kernel scaffold; baseline (speedup 1.0000x reference)
import jax
import jax.numpy as jnp
from jax.experimental import pallas as pl


def kernel(positions, boxvectors, node_attrs, W_embed, W_rbf, W1, w_out, neighbors, shift_idx):
    raise NotImplementedError("write your pallas kernel here")


# speedup vs baseline: 19.6419x; 19.6419x over previous
import jax
import jax.numpy as jnp
from jax import lax
from jax.experimental import pallas as pl
from jax.experimental.pallas import tpu as pltpu
from jax.experimental.pallas import tpu_sc as plsc

_N = 10000
_NSP = 10
_HID = 128
_NRBF = 8
_RMAX = 5.0
_LAM = 0.5
_ESCALE = 96.4853
_LSCALE = 10.0

_NW = 32
_EPW = 5120
_EPAD = _NW * _EPW
_B = 512
_NBLK = _EPW // _B
_AROWS = _N * _NSP
_ARPAD = 102400
_RPT = _ARPAD // 16
_CPT = _RPT // _B
_TAIL = _RPT % _B
_PI = 3.14159265358979


def _sc_edge_kernel(tbl, srcp, dstp, sxp, syp, szp, zrows,
                    a1_out, a2_out,
                    src_v, dst_v, sx_v, sy_v, sz_v,
                    rows_s_v, rows_d_v,
                    idxf_v, idxr_v, vals1_v, vals2_v,
                    sem, a1_sh, a2_sh):
    cid = lax.axis_index("c")
    sid = lax.axis_index("s")
    wid = cid * 16 + sid

    pltpu.sync_copy(zrows, vals1_v)
    r0 = sid * _RPT
    for c in range(_CPT):
        pltpu.sync_copy(vals1_v, a1_sh.at[pl.ds(r0 + c * _B, _B)])
        pltpu.sync_copy(vals1_v, a2_sh.at[pl.ds(r0 + c * _B, _B)])
    t0 = r0 + _CPT * _B
    pltpu.sync_copy(vals1_v.at[pl.ds(0, _TAIL)], a1_sh.at[pl.ds(t0, _TAIL)])
    pltpu.sync_copy(vals1_v.at[pl.ds(0, _TAIL)], a2_sh.at[pl.ds(t0, _TAIL)])
    plsc.subcore_barrier()

    lane = lax.iota(jnp.int32, 16)
    ebase = wid * _EPW
    col0 = lane * 0
    col1 = col0 + 1
    col2 = col0 + 2
    col3 = col0 + 3

    for blk in range(_NBLK):
        b0 = ebase + blk * _B
        pltpu.sync_copy(srcp.at[pl.ds(b0, _B)], src_v)
        pltpu.sync_copy(dstp.at[pl.ds(b0, _B)], dst_v)
        pltpu.sync_copy(sxp.at[pl.ds(b0, _B)], sx_v)
        pltpu.sync_copy(syp.at[pl.ds(b0, _B)], sy_v)
        pltpu.sync_copy(szp.at[pl.ds(b0, _B)], sz_v)
        pltpu.async_copy(tbl.at[src_v], rows_s_v, sem).wait()
        pltpu.async_copy(tbl.at[dst_v], rows_d_v, sem).wait()

        def group(g, carry):
            o = g * 16
            s_i = src_v[pl.ds(o, 16)]
            d_i = dst_v[pl.ds(o, 16)]
            sxf = sx_v[pl.ds(o, 16)]
            syf = sy_v[pl.ds(o, 16)]
            szf = sz_v[pl.ds(o, 16)]
            lrow = o + lane
            xs = plsc.load_gather(rows_s_v, [lrow, col0])
            ys = plsc.load_gather(rows_s_v, [lrow, col1])
            zs_ = plsc.load_gather(rows_s_v, [lrow, col2])
            zsrc = plsc.load_gather(rows_s_v, [lrow, col3]).astype(jnp.int32)
            xd = plsc.load_gather(rows_d_v, [lrow, col0])
            yd = plsc.load_gather(rows_d_v, [lrow, col1])
            zd_ = plsc.load_gather(rows_d_v, [lrow, col2])
            zdst = plsc.load_gather(rows_d_v, [lrow, col3]).astype(jnp.int32)

            vx = xs - xd + sxf
            vy = ys - yd + syf
            vz = zs_ - zd_ + szf
            r2 = vx * vx + vy * vy + vz * vz + 1e-9

            ii = jnp.int32(0x5F3759DF) - (plsc.bitcast(r2, jnp.int32) >> 1)
            y = plsc.bitcast(ii, jnp.float32)
            y = y * (1.5 - 0.5 * r2 * y * y)
            y = y * (1.5 - 0.5 * r2 * y * y)
            y = y * (1.5 - 0.5 * r2 * y * y)
            r = r2 * y

            theta = jnp.minimum(r * jnp.float32(_PI / _RMAX), jnp.float32(_PI))
            t = theta - jnp.float32(_PI / 2)
            t2 = t * t
            s1b = 1.0 + t2 * (-1.0 / 2 + t2 * (1.0 / 24 + t2 * (
                -1.0 / 720 + t2 * (1.0 / 40320 - t2 * (1.0 / 3628800)))))
            c1b = -t * (1.0 + t2 * (-1.0 / 6 + t2 * (1.0 / 120 + t2 * (
                -1.0 / 5040 + t2 * (1.0 / 362880)))))
            h2 = theta * theta
            sin_s = theta * (1.0 + h2 * (-1.0 / 6 + h2 * (
                1.0 / 120 - h2 * (1.0 / 5040))))
            cos_s = 1.0 + h2 * (-1.0 / 2 + h2 * (1.0 / 24 + h2 * (
                -1.0 / 720 + h2 * (1.0 / 40320))))
            small = theta < 1.0
            s1 = jnp.where(small, sin_s, s1b)
            c1 = jnp.where(small, cos_s, c1b)

            x = jnp.minimum(r * jnp.float32(1.0 / _RMAX), 1.0)
            env = 1.0 + x * x * x * (-10.0 + x * (15.0 - 6.0 * x))
            one = jnp.float32(1.0)
            zero = jnp.float32(0.0)
            m1 = jnp.where((s_i != 0) & (d_i != 0), one, zero)
            m2 = jnp.where((s_i != 1) & (d_i != 1), one, zero)
            sc1 = y * env * m1
            sc2 = y * env * m2

            idxf_v[pl.ds(o, 16)] = d_i * _NSP + zsrc
            idxr_v[pl.ds(o, 16)] = s_i * _NSP + zdst

            tc = 2.0 * c1
            sk_prev = s1
            sk = tc * s1
            plsc.store_scatter(vals1_v, [lrow, col0], s1 * sc1)
            plsc.store_scatter(vals2_v, [lrow, col0], s1 * sc2)
            for k in range(1, _NRBF):
                colk = col0 + k
                plsc.store_scatter(vals1_v, [lrow, colk], sk * sc1)
                plsc.store_scatter(vals2_v, [lrow, colk], sk * sc2)
                sk_new = tc * sk - sk_prev
                sk_prev = sk
                sk = sk_new
            return carry

        lax.fori_loop(0, _B // 16, group, 0)

        pltpu.sync_copy(vals1_v, a1_sh.at[idxf_v], add=True)
        pltpu.sync_copy(vals1_v, a1_sh.at[idxr_v], add=True)
        pltpu.sync_copy(vals2_v, a2_sh.at[idxf_v], add=True)
        pltpu.sync_copy(vals2_v, a2_sh.at[idxr_v], add=True)

    plsc.subcore_barrier()

    for c in range(_CPT):
        rr = r0 + c * _B
        pltpu.sync_copy(a1_sh.at[pl.ds(rr, _B)], vals1_v)
        pltpu.sync_copy(vals1_v, a1_out.at[cid, pl.ds(rr, _B)])
        pltpu.sync_copy(a2_sh.at[pl.ds(rr, _B)], vals2_v)
        pltpu.sync_copy(vals2_v, a2_out.at[cid, pl.ds(rr, _B)])
    pltpu.sync_copy(a1_sh.at[pl.ds(t0, _TAIL)], vals1_v.at[pl.ds(0, _TAIL)])
    pltpu.sync_copy(vals1_v.at[pl.ds(0, _TAIL)], a1_out.at[cid, pl.ds(t0, _TAIL)])
    pltpu.sync_copy(a2_sh.at[pl.ds(t0, _TAIL)], vals2_v.at[pl.ds(0, _TAIL)])
    pltpu.sync_copy(vals2_v.at[pl.ds(0, _TAIL)], a2_out.at[cid, pl.ds(t0, _TAIL)])


def _run_sc(tbl, srcp, dstp, sxp, syp, szp, zrows):
    mesh = plsc.VectorSubcoreMesh(core_axis_name="c", subcore_axis_name="s")
    f = pl.kernel(
        _sc_edge_kernel,
        out_type=(
            jax.ShapeDtypeStruct((2, _ARPAD, _NRBF), jnp.float32),
            jax.ShapeDtypeStruct((2, _ARPAD, _NRBF), jnp.float32),
        ),
        mesh=mesh,
        compiler_params=pltpu.CompilerParams(
            needs_layout_passes=False, use_tc_tiling_on_sc=False),
        scratch_types=[
            pltpu.VMEM((_B,), jnp.int32),
            pltpu.VMEM((_B,), jnp.int32),
            pltpu.VMEM((_B,), jnp.float32),
            pltpu.VMEM((_B,), jnp.float32),
            pltpu.VMEM((_B,), jnp.float32),
            pltpu.VMEM((_B, _NRBF), jnp.float32),
            pltpu.VMEM((_B, _NRBF), jnp.float32),
            pltpu.VMEM((_B,), jnp.int32),
            pltpu.VMEM((_B,), jnp.int32),
            pltpu.VMEM((_B, _NRBF), jnp.float32),
            pltpu.VMEM((_B, _NRBF), jnp.float32),
            pltpu.SemaphoreType.DMA,
            pltpu.VMEM_SHARED((_ARPAD, _NRBF), jnp.float32),
            pltpu.VMEM_SHARED((_ARPAD, _NRBF), jnp.float32),
        ],
    )
    return f(tbl, srcp, dstp, sxp, syp, szp, zrows)


_NP = 10240
_BN = 2048


def _tc_body(a1_ref, a2_ref, we_ref, wr_ref, w1_ref, wo_ref, out_ref, u_ref):
    i = pl.program_id(0)

    @pl.when(i == 0)
    def _init():
        t = (we_ref[:][:, None, :] * wr_ref[:][None, :, :]).reshape(
            _NSP * _NRBF, _HID)
        u_ref[:] = jnp.dot(t, w1_ref[:], preferred_element_type=jnp.float32,
                           precision=jax.lax.Precision.HIGHEST)
        out_ref[0, 0] = 0.0

    u = u_ref[:]
    a1 = a1_ref[0] + a1_ref[1]
    a2 = a2_ref[0] + a2_ref[1]
    h1 = jnp.tanh(jnp.dot(a1, u, preferred_element_type=jnp.float32,
                          precision=jax.lax.Precision.HIGHEST))
    h2 = jnp.tanh(jnp.dot(a2, u, preferred_element_type=jnp.float32,
                          precision=jax.lax.Precision.HIGHEST))
    ne = _LAM * jnp.dot(h1, wo_ref[:], preferred_element_type=jnp.float32,
                        precision=jax.lax.Precision.HIGHEST) \
        + (1.0 - _LAM) * jnp.dot(h2, wo_ref[:], preferred_element_type=jnp.float32,
                                 precision=jax.lax.Precision.HIGHEST)
    out_ref[0, 0] += _ESCALE * jnp.sum(ne)


def _run_tc(a1p, a2p, w_embed, w_rbf, w1, wo2):
    grid = (_NP // _BN,)
    return pl.pallas_call(
        _tc_body,
        grid=grid,
        in_specs=[
            pl.BlockSpec((2, _BN, _NSP * _NRBF), lambda i: (0, i, 0)),
            pl.BlockSpec((2, _BN, _NSP * _NRBF), lambda i: (0, i, 0)),
            pl.BlockSpec((_NSP, _HID), lambda i: (0, 0)),
            pl.BlockSpec((_NRBF, _HID), lambda i: (0, 0)),
            pl.BlockSpec((_HID, _HID), lambda i: (0, 0)),
            pl.BlockSpec((_HID, 1), lambda i: (0, 0)),
        ],
        out_specs=pl.BlockSpec(memory_space=pltpu.SMEM),
        out_shape=jax.ShapeDtypeStruct((1, 1), jnp.float32),
        scratch_shapes=[pltpu.VMEM((_NSP * _NRBF, _HID), jnp.float32)],
    )(a1p, a2p, w_embed, w_rbf, w1, wo2)


def kernel(positions, boxvectors, node_attrs, W_embed, W_rbf, W1, w_out,
           neighbors, shift_idx):
    pos = positions.astype(jnp.float32) * _LSCALE
    cell = boxvectors.astype(jnp.float32) * _LSCALE
    spec = jnp.argmax(node_attrs, axis=1).astype(jnp.float32)
    tbl = jnp.concatenate(
        [pos, spec[:, None], jnp.zeros((_N, 4), jnp.float32)], axis=1)

    src = neighbors[0].astype(jnp.int32)
    dst = neighbors[1].astype(jnp.int32)
    shf = shift_idx.astype(jnp.float32) @ cell
    npad = _EPAD - src.shape[0]
    srcp = jnp.concatenate([src, jnp.zeros((npad,), jnp.int32)])
    dstp = jnp.concatenate([dst, jnp.zeros((npad,), jnp.int32)])
    big = jnp.full((npad,), 30000.0, jnp.float32)
    sxp = jnp.concatenate([shf[:, 0] + 0.0, big])
    syp = jnp.concatenate([shf[:, 1] + 0.0, big])
    szp = jnp.concatenate([shf[:, 2] + 0.0, big])
    zrows = jnp.zeros((_B, _NRBF), jnp.float32)

    a1p, a2p = _run_sc(tbl, srcp, dstp, sxp, syp, szp, zrows)
    a1p = a1p.reshape(2, _NP, _NSP * _NRBF)
    a2p = a2p.reshape(2, _NP, _NSP * _NRBF)
    out = _run_tc(a1p, a2p, W_embed, W_rbf, W1, w_out.reshape(_HID, 1))
    return out[0, 0]


# speedup vs baseline: 20.0367x; 1.0201x over previous
import jax
import jax.numpy as jnp
from jax import lax
from jax.experimental import pallas as pl
from jax.experimental.pallas import tpu as pltpu
from jax.experimental.pallas import tpu_sc as plsc

_N = 10000
_NSP = 10
_HID = 128
_NRBF = 8
_RMAX = 5.0
_LAM = 0.5
_ESCALE = 96.4853
_LSCALE = 10.0

_NW = 32
_EPW = 5120
_EPAD = _NW * _EPW
_B = 512
_NBLK = _EPW // _B
_AROWS = _N * _NSP
_ARPAD = 102400
_RPT = _ARPAD // 16
_CPT = _RPT // _B
_TAIL = _RPT % _B
_PI = 3.14159265358979


def _sc_edge_kernel(tbl, srcp, dstp, sxp, syp, szp, zrows,
                    a1_out, a2_out,
                    src_v, dst_v, sx_v, sy_v, sz_v,
                    rows_s_v, rows_d_v,
                    idxf_v, idxr_v, vals1_v, vals2_v,
                    sem, a1_sh, a2_sh):
    cid = lax.axis_index("c")
    sid = lax.axis_index("s")
    wid = cid * 16 + sid

    pltpu.sync_copy(zrows, vals1_v)
    r0 = sid * _RPT
    for c in range(_CPT):
        pltpu.sync_copy(vals1_v, a1_sh.at[pl.ds(r0 + c * _B, _B)])
        pltpu.sync_copy(vals1_v, a2_sh.at[pl.ds(r0 + c * _B, _B)])
    t0 = r0 + _CPT * _B
    pltpu.sync_copy(vals1_v.at[pl.ds(0, _TAIL)], a1_sh.at[pl.ds(t0, _TAIL)])
    pltpu.sync_copy(vals1_v.at[pl.ds(0, _TAIL)], a2_sh.at[pl.ds(t0, _TAIL)])
    plsc.subcore_barrier()

    lane = lax.iota(jnp.int32, 16)
    ebase = wid * _EPW
    col0 = lane * 0
    col1 = col0 + 1
    col2 = col0 + 2
    col3 = col0 + 3

    for blk in range(_NBLK):
        b0 = ebase + blk * _B
        pltpu.sync_copy(srcp.at[pl.ds(b0, _B)], src_v)
        pltpu.sync_copy(dstp.at[pl.ds(b0, _B)], dst_v)
        pltpu.sync_copy(sxp.at[pl.ds(b0, _B)], sx_v)
        pltpu.sync_copy(syp.at[pl.ds(b0, _B)], sy_v)
        pltpu.sync_copy(szp.at[pl.ds(b0, _B)], sz_v)
        pltpu.async_copy(tbl.at[src_v], rows_s_v, sem).wait()
        pltpu.async_copy(tbl.at[dst_v], rows_d_v, sem).wait()

        def group(g, carry):
            o = g * 16
            s_i = src_v[pl.ds(o, 16)]
            d_i = dst_v[pl.ds(o, 16)]
            sxf = sx_v[pl.ds(o, 16)]
            syf = sy_v[pl.ds(o, 16)]
            szf = sz_v[pl.ds(o, 16)]
            lrow = o + lane
            xs = plsc.load_gather(rows_s_v, [lrow, col0])
            ys = plsc.load_gather(rows_s_v, [lrow, col1])
            zs_ = plsc.load_gather(rows_s_v, [lrow, col2])
            zsrc = plsc.load_gather(rows_s_v, [lrow, col3]).astype(jnp.int32)
            xd = plsc.load_gather(rows_d_v, [lrow, col0])
            yd = plsc.load_gather(rows_d_v, [lrow, col1])
            zd_ = plsc.load_gather(rows_d_v, [lrow, col2])
            zdst = plsc.load_gather(rows_d_v, [lrow, col3]).astype(jnp.int32)

            vx = xs - xd + sxf
            vy = ys - yd + syf
            vz = zs_ - zd_ + szf
            r2 = vx * vx + vy * vy + vz * vz + 1e-9

            ii = jnp.int32(0x5F3759DF) - (plsc.bitcast(r2, jnp.int32) >> 1)
            y = plsc.bitcast(ii, jnp.float32)
            y = y * (1.5 - 0.5 * r2 * y * y)
            y = y * (1.5 - 0.5 * r2 * y * y)
            y = y * (1.5 - 0.5 * r2 * y * y)
            r = r2 * y

            theta = jnp.minimum(r * jnp.float32(_PI / _RMAX), jnp.float32(_PI))
            t = theta - jnp.float32(_PI / 2)
            t2 = t * t
            s1b = 1.0 + t2 * (-1.0 / 2 + t2 * (1.0 / 24 + t2 * (
                -1.0 / 720 + t2 * (1.0 / 40320 - t2 * (1.0 / 3628800)))))
            c1b = -t * (1.0 + t2 * (-1.0 / 6 + t2 * (1.0 / 120 + t2 * (
                -1.0 / 5040 + t2 * (1.0 / 362880)))))
            h2 = theta * theta
            sin_s = theta * (1.0 + h2 * (-1.0 / 6 + h2 * (
                1.0 / 120 - h2 * (1.0 / 5040))))
            cos_s = 1.0 + h2 * (-1.0 / 2 + h2 * (1.0 / 24 + h2 * (
                -1.0 / 720 + h2 * (1.0 / 40320))))
            small = theta < 1.0
            s1 = jnp.where(small, sin_s, s1b)
            c1 = jnp.where(small, cos_s, c1b)

            x = jnp.minimum(r * jnp.float32(1.0 / _RMAX), 1.0)
            env = 1.0 + x * x * x * (-10.0 + x * (15.0 - 6.0 * x))
            one = jnp.float32(1.0)
            zero = jnp.float32(0.0)
            m1 = jnp.where((s_i != 0) & (d_i != 0), one, zero)
            m2 = jnp.where((s_i != 1) & (d_i != 1), one, zero)
            scale = y * env

            def bf16r(v):
                u = plsc.bitcast(v, jnp.int32)
                lsb = (u >> 16) & 1
                return plsc.bitcast((u + 32767 + lsb) & jnp.int32(-65536),
                                    jnp.float32)

            idxf_v[pl.ds(o, 16)] = d_i * _NSP + zsrc
            idxr_v[pl.ds(o, 16)] = s_i * _NSP + zdst

            tc = 2.0 * c1
            sk_prev = s1
            sk = tc * s1
            v0 = bf16r(s1 * scale)
            plsc.store_scatter(vals1_v, [lrow, col0], v0 * m1)
            plsc.store_scatter(vals2_v, [lrow, col0], v0 * m2)
            for k in range(1, _NRBF):
                colk = col0 + k
                vk = bf16r(sk * scale)
                plsc.store_scatter(vals1_v, [lrow, colk], vk * m1)
                plsc.store_scatter(vals2_v, [lrow, colk], vk * m2)
                sk_new = tc * sk - sk_prev
                sk_prev = sk
                sk = sk_new
            return carry

        lax.fori_loop(0, _B // 16, group, 0)

        pltpu.sync_copy(vals1_v, a1_sh.at[idxf_v], add=True)
        pltpu.sync_copy(vals1_v, a1_sh.at[idxr_v], add=True)
        pltpu.sync_copy(vals2_v, a2_sh.at[idxf_v], add=True)
        pltpu.sync_copy(vals2_v, a2_sh.at[idxr_v], add=True)

    plsc.subcore_barrier()

    for c in range(_CPT):
        rr = r0 + c * _B
        pltpu.sync_copy(a1_sh.at[pl.ds(rr, _B)], vals1_v)
        pltpu.sync_copy(vals1_v, a1_out.at[cid, pl.ds(rr, _B)])
        pltpu.sync_copy(a2_sh.at[pl.ds(rr, _B)], vals2_v)
        pltpu.sync_copy(vals2_v, a2_out.at[cid, pl.ds(rr, _B)])
    pltpu.sync_copy(a1_sh.at[pl.ds(t0, _TAIL)], vals1_v.at[pl.ds(0, _TAIL)])
    pltpu.sync_copy(vals1_v.at[pl.ds(0, _TAIL)], a1_out.at[cid, pl.ds(t0, _TAIL)])
    pltpu.sync_copy(a2_sh.at[pl.ds(t0, _TAIL)], vals2_v.at[pl.ds(0, _TAIL)])
    pltpu.sync_copy(vals2_v.at[pl.ds(0, _TAIL)], a2_out.at[cid, pl.ds(t0, _TAIL)])


def _run_sc(tbl, srcp, dstp, sxp, syp, szp, zrows):
    mesh = plsc.VectorSubcoreMesh(core_axis_name="c", subcore_axis_name="s")
    f = pl.kernel(
        _sc_edge_kernel,
        out_type=(
            jax.ShapeDtypeStruct((2, _ARPAD, _NRBF), jnp.float32),
            jax.ShapeDtypeStruct((2, _ARPAD, _NRBF), jnp.float32),
        ),
        mesh=mesh,
        compiler_params=pltpu.CompilerParams(
            needs_layout_passes=False, use_tc_tiling_on_sc=False),
        scratch_types=[
            pltpu.VMEM((_B,), jnp.int32),
            pltpu.VMEM((_B,), jnp.int32),
            pltpu.VMEM((_B,), jnp.float32),
            pltpu.VMEM((_B,), jnp.float32),
            pltpu.VMEM((_B,), jnp.float32),
            pltpu.VMEM((_B, _NRBF), jnp.float32),
            pltpu.VMEM((_B, _NRBF), jnp.float32),
            pltpu.VMEM((_B,), jnp.int32),
            pltpu.VMEM((_B,), jnp.int32),
            pltpu.VMEM((_B, _NRBF), jnp.float32),
            pltpu.VMEM((_B, _NRBF), jnp.float32),
            pltpu.SemaphoreType.DMA,
            pltpu.VMEM_SHARED((_ARPAD, _NRBF), jnp.float32),
            pltpu.VMEM_SHARED((_ARPAD, _NRBF), jnp.float32),
        ],
    )
    return f(tbl, srcp, dstp, sxp, syp, szp, zrows)


_NP = 10240
_BN = 2048


def _tc_body(a1_ref, a2_ref, we_ref, wr_ref, w1_ref, wo_ref, out_ref, u_ref):
    i = pl.program_id(0)

    @pl.when(i == 0)
    def _init():
        web = we_ref[:].astype(jnp.bfloat16).astype(jnp.float32)
        wrb = wr_ref[:].astype(jnp.bfloat16).astype(jnp.float32)
        u_ref[:] = (web[:, None, :] * wrb[None, :, :]).reshape(
            _NSP * _NRBF, _HID)
        out_ref[0, 0] = 0.0

    u = u_ref[:]
    a1 = a1_ref[0] + a1_ref[1]
    a2 = a2_ref[0] + a2_ref[1]
    m1 = jnp.dot(a1, u, preferred_element_type=jnp.float32,
                 precision=jax.lax.Precision.HIGHEST)
    m2 = jnp.dot(a2, u, preferred_element_type=jnp.float32,
                 precision=jax.lax.Precision.HIGHEST)
    h1 = jnp.tanh(jnp.dot(m1, w1_ref[:], preferred_element_type=jnp.float32))
    h2 = jnp.tanh(jnp.dot(m2, w1_ref[:], preferred_element_type=jnp.float32))
    ne = _LAM * jnp.dot(h1, wo_ref[:], preferred_element_type=jnp.float32) \
        + (1.0 - _LAM) * jnp.dot(h2, wo_ref[:], preferred_element_type=jnp.float32)
    out_ref[0, 0] += _ESCALE * jnp.sum(ne)


def _run_tc(a1p, a2p, w_embed, w_rbf, w1, wo2):
    grid = (_NP // _BN,)
    return pl.pallas_call(
        _tc_body,
        grid=grid,
        in_specs=[
            pl.BlockSpec((2, _BN, _NSP * _NRBF), lambda i: (0, i, 0)),
            pl.BlockSpec((2, _BN, _NSP * _NRBF), lambda i: (0, i, 0)),
            pl.BlockSpec((_NSP, _HID), lambda i: (0, 0)),
            pl.BlockSpec((_NRBF, _HID), lambda i: (0, 0)),
            pl.BlockSpec((_HID, _HID), lambda i: (0, 0)),
            pl.BlockSpec((_HID, 1), lambda i: (0, 0)),
        ],
        out_specs=pl.BlockSpec(memory_space=pltpu.SMEM),
        out_shape=jax.ShapeDtypeStruct((1, 1), jnp.float32),
        scratch_shapes=[pltpu.VMEM((_NSP * _NRBF, _HID), jnp.float32)],
    )(a1p, a2p, w_embed, w_rbf, w1, wo2)


def kernel(positions, boxvectors, node_attrs, W_embed, W_rbf, W1, w_out,
           neighbors, shift_idx):
    pos = positions.astype(jnp.float32) * _LSCALE
    cell = boxvectors.astype(jnp.float32) * _LSCALE
    spec = jnp.argmax(node_attrs, axis=1).astype(jnp.float32)
    tbl = jnp.concatenate(
        [pos, spec[:, None], jnp.zeros((_N, 4), jnp.float32)], axis=1)

    src = neighbors[0].astype(jnp.int32)
    dst = neighbors[1].astype(jnp.int32)
    shf = shift_idx.astype(jnp.float32) @ cell
    npad = _EPAD - src.shape[0]
    srcp = jnp.concatenate([src, jnp.zeros((npad,), jnp.int32)])
    dstp = jnp.concatenate([dst, jnp.zeros((npad,), jnp.int32)])
    big = jnp.full((npad,), 30000.0, jnp.float32)
    sxp = jnp.concatenate([shf[:, 0] + 0.0, big])
    syp = jnp.concatenate([shf[:, 1] + 0.0, big])
    szp = jnp.concatenate([shf[:, 2] + 0.0, big])
    zrows = jnp.zeros((_B, _NRBF), jnp.float32)

    a1p, a2p = _run_sc(tbl, srcp, dstp, sxp, syp, szp, zrows)
    a1p = a1p.reshape(2, _NP, _NSP * _NRBF)
    a2p = a2p.reshape(2, _NP, _NSP * _NRBF)
    out = _run_tc(a1p, a2p, W_embed, W_rbf, W1, w_out.reshape(_HID, 1))
    return out[0, 0]


# speedup vs baseline: 23.9353x; 1.1946x over previous
import jax
import jax.numpy as jnp
from jax import lax
from jax.experimental import pallas as pl
from jax.experimental.pallas import tpu as pltpu
from jax.experimental.pallas import tpu_sc as plsc

_N = 10000
_NSP = 10
_HID = 128
_NRBF = 8
_RMAX = 5.0
_LAM = 0.5
_ESCALE = 96.4853
_LSCALE = 10.0

_NW = 32
_EPW = 5120
_EPAD = _NW * _EPW
_B = 512
_NBLK = _EPW // _B
_AROWS = _N * _NSP
_ARPAD = 102400
_RPT = _ARPAD // 16
_CPT = _RPT // _B
_TAIL = _RPT % _B
_PI = 3.14159265358979


def _sc_edge_kernel(tbl, srcp, dstp, sxp, syp, szp, zrows,
                    a1_out, a2_out,
                    src_v, dst_v, sx_v, sy_v, sz_v,
                    rows_s_v, rows_d_v,
                    idxf_v, idxr_v, vals1_v, vals2_v,
                    sem, sem2, sem3, a1_sh, a2_sh):
    cid = lax.axis_index("c")
    sid = lax.axis_index("s")
    wid = cid * 16 + sid

    pltpu.sync_copy(zrows, vals1_v)
    r0 = sid * _RPT
    for c in range(_CPT):
        pltpu.sync_copy(vals1_v, a1_sh.at[pl.ds(r0 + c * _B, _B)])
        pltpu.sync_copy(vals1_v, a2_sh.at[pl.ds(r0 + c * _B, _B)])
    t0 = r0 + _CPT * _B
    pltpu.sync_copy(vals1_v.at[pl.ds(0, _TAIL)], a1_sh.at[pl.ds(t0, _TAIL)])
    pltpu.sync_copy(vals1_v.at[pl.ds(0, _TAIL)], a2_sh.at[pl.ds(t0, _TAIL)])
    plsc.subcore_barrier()

    lane = lax.iota(jnp.int32, 16)
    ebase = wid * _EPW
    col0 = lane * 0
    col1 = col0 + 1
    col2 = col0 + 2
    col3 = col0 + 3

    pend = []
    for blk in range(_NBLK):
        b0 = ebase + blk * _B
        loads = [
            pltpu.async_copy(srcp.at[pl.ds(b0, _B)], src_v, sem),
            pltpu.async_copy(dstp.at[pl.ds(b0, _B)], dst_v, sem),
            pltpu.async_copy(sxp.at[pl.ds(b0, _B)], sx_v, sem),
            pltpu.async_copy(syp.at[pl.ds(b0, _B)], sy_v, sem),
            pltpu.async_copy(szp.at[pl.ds(b0, _B)], sz_v, sem),
        ]
        for h in loads:
            h.wait()
        g1 = pltpu.async_copy(tbl.at[src_v], rows_s_v, sem2)
        g2 = pltpu.async_copy(tbl.at[dst_v], rows_d_v, sem2)
        g1.wait()
        g2.wait()
        for h in pend:
            h.wait()

        def group(g, carry):
            o = g * 16
            s_i = src_v[pl.ds(o, 16)]
            d_i = dst_v[pl.ds(o, 16)]
            sxf = sx_v[pl.ds(o, 16)]
            syf = sy_v[pl.ds(o, 16)]
            szf = sz_v[pl.ds(o, 16)]
            lrow = o + lane
            xs = plsc.load_gather(rows_s_v, [lrow, col0])
            ys = plsc.load_gather(rows_s_v, [lrow, col1])
            zs_ = plsc.load_gather(rows_s_v, [lrow, col2])
            zsrc = plsc.load_gather(rows_s_v, [lrow, col3]).astype(jnp.int32)
            xd = plsc.load_gather(rows_d_v, [lrow, col0])
            yd = plsc.load_gather(rows_d_v, [lrow, col1])
            zd_ = plsc.load_gather(rows_d_v, [lrow, col2])
            zdst = plsc.load_gather(rows_d_v, [lrow, col3]).astype(jnp.int32)

            vx = xs - xd + sxf
            vy = ys - yd + syf
            vz = zs_ - zd_ + szf
            r2 = vx * vx + vy * vy + vz * vz + 1e-9

            ii = jnp.int32(0x5F3759DF) - (plsc.bitcast(r2, jnp.int32) >> 1)
            y = plsc.bitcast(ii, jnp.float32)
            y = y * (1.5 - 0.5 * r2 * y * y)
            y = y * (1.5 - 0.5 * r2 * y * y)
            y = y * (1.5 - 0.5 * r2 * y * y)
            r = r2 * y

            theta = jnp.minimum(r * jnp.float32(_PI / _RMAX), jnp.float32(_PI))
            t = theta - jnp.float32(_PI / 2)
            t2 = t * t
            s1b = 1.0 + t2 * (-1.0 / 2 + t2 * (1.0 / 24 + t2 * (
                -1.0 / 720 + t2 * (1.0 / 40320 - t2 * (1.0 / 3628800)))))
            c1b = -t * (1.0 + t2 * (-1.0 / 6 + t2 * (1.0 / 120 + t2 * (
                -1.0 / 5040 + t2 * (1.0 / 362880)))))
            h2 = theta * theta
            sin_s = theta * (1.0 + h2 * (-1.0 / 6 + h2 * (
                1.0 / 120 - h2 * (1.0 / 5040))))
            cos_s = 1.0 + h2 * (-1.0 / 2 + h2 * (1.0 / 24 + h2 * (
                -1.0 / 720 + h2 * (1.0 / 40320))))
            small = theta < 1.0
            s1 = jnp.where(small, sin_s, s1b)
            c1 = jnp.where(small, cos_s, c1b)

            x = jnp.minimum(r * jnp.float32(1.0 / _RMAX), 1.0)
            env = 1.0 + x * x * x * (-10.0 + x * (15.0 - 6.0 * x))
            one = jnp.float32(1.0)
            zero = jnp.float32(0.0)
            m1 = jnp.where((s_i != 0) & (d_i != 0), one, zero)
            m2 = jnp.where((s_i != 1) & (d_i != 1), one, zero)
            scale = y * env

            def bf16r(v):
                u = plsc.bitcast(v, jnp.int32)
                lsb = (u >> 16) & 1
                return plsc.bitcast((u + 32767 + lsb) & jnp.int32(-65536),
                                    jnp.float32)

            idxf_v[pl.ds(o, 16)] = d_i * _NSP + zsrc
            idxr_v[pl.ds(o, 16)] = s_i * _NSP + zdst

            tc = 2.0 * c1
            sk_prev = s1
            sk = tc * s1
            v0 = bf16r(s1 * scale)
            plsc.store_scatter(vals1_v, [lrow, col0], v0 * m1)
            plsc.store_scatter(vals2_v, [lrow, col0], v0 * m2)
            for k in range(1, _NRBF):
                colk = col0 + k
                vk = bf16r(sk * scale)
                plsc.store_scatter(vals1_v, [lrow, colk], vk * m1)
                plsc.store_scatter(vals2_v, [lrow, colk], vk * m2)
                sk_new = tc * sk - sk_prev
                sk_prev = sk
                sk = sk_new
            return carry

        lax.fori_loop(0, _B // 16, group, 0)

        pend = [
            pltpu.async_copy(vals1_v, a1_sh.at[idxf_v], sem3, add=True),
            pltpu.async_copy(vals1_v, a1_sh.at[idxr_v], sem3, add=True),
            pltpu.async_copy(vals2_v, a2_sh.at[idxf_v], sem3, add=True),
            pltpu.async_copy(vals2_v, a2_sh.at[idxr_v], sem3, add=True),
        ]

    for h in pend:
        h.wait()
    plsc.subcore_barrier()

    for c in range(_CPT):
        rr = r0 + c * _B
        pltpu.sync_copy(a1_sh.at[pl.ds(rr, _B)], vals1_v)
        pltpu.sync_copy(vals1_v, a1_out.at[cid, pl.ds(rr, _B)])
        pltpu.sync_copy(a2_sh.at[pl.ds(rr, _B)], vals2_v)
        pltpu.sync_copy(vals2_v, a2_out.at[cid, pl.ds(rr, _B)])
    pltpu.sync_copy(a1_sh.at[pl.ds(t0, _TAIL)], vals1_v.at[pl.ds(0, _TAIL)])
    pltpu.sync_copy(vals1_v.at[pl.ds(0, _TAIL)], a1_out.at[cid, pl.ds(t0, _TAIL)])
    pltpu.sync_copy(a2_sh.at[pl.ds(t0, _TAIL)], vals2_v.at[pl.ds(0, _TAIL)])
    pltpu.sync_copy(vals2_v.at[pl.ds(0, _TAIL)], a2_out.at[cid, pl.ds(t0, _TAIL)])


def _run_sc(tbl, srcp, dstp, sxp, syp, szp, zrows):
    mesh = plsc.VectorSubcoreMesh(core_axis_name="c", subcore_axis_name="s")
    f = pl.kernel(
        _sc_edge_kernel,
        out_type=(
            jax.ShapeDtypeStruct((2, _ARPAD, _NRBF), jnp.float32),
            jax.ShapeDtypeStruct((2, _ARPAD, _NRBF), jnp.float32),
        ),
        mesh=mesh,
        compiler_params=pltpu.CompilerParams(
            needs_layout_passes=False, use_tc_tiling_on_sc=False),
        scratch_types=[
            pltpu.VMEM((_B,), jnp.int32),
            pltpu.VMEM((_B,), jnp.int32),
            pltpu.VMEM((_B,), jnp.float32),
            pltpu.VMEM((_B,), jnp.float32),
            pltpu.VMEM((_B,), jnp.float32),
            pltpu.VMEM((_B, _NRBF), jnp.float32),
            pltpu.VMEM((_B, _NRBF), jnp.float32),
            pltpu.VMEM((_B,), jnp.int32),
            pltpu.VMEM((_B,), jnp.int32),
            pltpu.VMEM((_B, _NRBF), jnp.float32),
            pltpu.VMEM((_B, _NRBF), jnp.float32),
            pltpu.SemaphoreType.DMA,
            pltpu.SemaphoreType.DMA,
            pltpu.SemaphoreType.DMA,
            pltpu.VMEM_SHARED((_ARPAD, _NRBF), jnp.float32),
            pltpu.VMEM_SHARED((_ARPAD, _NRBF), jnp.float32),
        ],
    )
    return f(tbl, srcp, dstp, sxp, syp, szp, zrows)


_NP = 10240
_BN = 2048


def _tc_body(a1_ref, a2_ref, we_ref, wr_ref, w1_ref, wo_ref, out_ref, u_ref):
    i = pl.program_id(0)

    @pl.when(i == 0)
    def _init():
        web = we_ref[:].astype(jnp.bfloat16).astype(jnp.float32)
        wrb = wr_ref[:].astype(jnp.bfloat16).astype(jnp.float32)
        u_ref[:] = (web[:, None, :] * wrb[None, :, :]).reshape(
            _NSP * _NRBF, _HID)
        out_ref[0, 0] = 0.0

    u = u_ref[:]
    a1 = a1_ref[0] + a1_ref[1]
    a2 = a2_ref[0] + a2_ref[1]
    m1 = jnp.dot(a1, u, preferred_element_type=jnp.float32,
                 precision=jax.lax.Precision.HIGHEST)
    m2 = jnp.dot(a2, u, preferred_element_type=jnp.float32,
                 precision=jax.lax.Precision.HIGHEST)
    h1 = jnp.tanh(jnp.dot(m1, w1_ref[:], preferred_element_type=jnp.float32))
    h2 = jnp.tanh(jnp.dot(m2, w1_ref[:], preferred_element_type=jnp.float32))
    ne = _LAM * jnp.dot(h1, wo_ref[:], preferred_element_type=jnp.float32) \
        + (1.0 - _LAM) * jnp.dot(h2, wo_ref[:], preferred_element_type=jnp.float32)
    out_ref[0, 0] += _ESCALE * jnp.sum(ne)


def _run_tc(a1p, a2p, w_embed, w_rbf, w1, wo2):
    grid = (_NP // _BN,)
    return pl.pallas_call(
        _tc_body,
        grid=grid,
        in_specs=[
            pl.BlockSpec((2, _BN, _NSP * _NRBF), lambda i: (0, i, 0)),
            pl.BlockSpec((2, _BN, _NSP * _NRBF), lambda i: (0, i, 0)),
            pl.BlockSpec((_NSP, _HID), lambda i: (0, 0)),
            pl.BlockSpec((_NRBF, _HID), lambda i: (0, 0)),
            pl.BlockSpec((_HID, _HID), lambda i: (0, 0)),
            pl.BlockSpec((_HID, 1), lambda i: (0, 0)),
        ],
        out_specs=pl.BlockSpec(memory_space=pltpu.SMEM),
        out_shape=jax.ShapeDtypeStruct((1, 1), jnp.float32),
        scratch_shapes=[pltpu.VMEM((_NSP * _NRBF, _HID), jnp.float32)],
    )(a1p, a2p, w_embed, w_rbf, w1, wo2)


def kernel(positions, boxvectors, node_attrs, W_embed, W_rbf, W1, w_out,
           neighbors, shift_idx):
    pos = positions.astype(jnp.float32) * _LSCALE
    cell = boxvectors.astype(jnp.float32) * _LSCALE
    spec = jnp.argmax(node_attrs, axis=1).astype(jnp.float32)
    tbl = jnp.concatenate(
        [pos, spec[:, None], jnp.zeros((_N, 4), jnp.float32)], axis=1)

    src = neighbors[0].astype(jnp.int32)
    dst = neighbors[1].astype(jnp.int32)
    shf = shift_idx.astype(jnp.float32) @ cell
    npad = _EPAD - src.shape[0]
    srcp = jnp.concatenate([src, jnp.zeros((npad,), jnp.int32)])
    dstp = jnp.concatenate([dst, jnp.zeros((npad,), jnp.int32)])
    big = jnp.full((npad,), 30000.0, jnp.float32)
    sxp = jnp.concatenate([shf[:, 0] + 0.0, big])
    syp = jnp.concatenate([shf[:, 1] + 0.0, big])
    szp = jnp.concatenate([shf[:, 2] + 0.0, big])
    zrows = jnp.zeros((_B, _NRBF), jnp.float32)

    a1p, a2p = _run_sc(tbl, srcp, dstp, sxp, syp, szp, zrows)
    a1p = a1p.reshape(2, _NP, _NSP * _NRBF)
    a2p = a2p.reshape(2, _NP, _NSP * _NRBF)
    out = _run_tc(a1p, a2p, W_embed, W_rbf, W1, w_out.reshape(_HID, 1))
    return out[0, 0]
